# baseline (speedup 1.0000x reference)
import jax
import jax.numpy as jnp
from jax.experimental import pallas as pl

_NUM_NODES = [6890, 3445, 1723, 862, 431, 216, 108, 54, 27]


def _cheb_conv(x, w, row, col, norm):
    out = jnp.einsum('bnf,fg->bng', x, w[0])
    if w.shape[0] > 1:
        msg = x[:, col, :] * norm[None, :, None]
        tx1 = jnp.zeros_like(x).at[:, row, :].add(msg)
        out = out + jnp.einsum('bnf,fg->bng', tx1, w[1])
    return out


def _pool(x, up):
    up_row, up_col, up_val, n_out = up
    n_static = up_row.shape[0]
    gathered = x[:, up_col, :] * up_val[None, :, None]
    out = jnp.zeros((x.shape[0], n_static, x.shape[2]), dtype=x.dtype).at[:, up_row, :].add(gathered)
    return out + (jnp.asarray(n_out) - n_static).astype(x.dtype)


def _group_norm(x, gamma, beta, num_groups=32, eps=1e-5):
    B, C, N = x.shape
    xg = x.reshape(B, num_groups, C // num_groups, N)
    mean = jnp.mean(xg, axis=(2, 3), keepdims=True)
    var = jnp.var(xg, axis=(2, 3), keepdims=True)
    xg = (xg - mean) * jax.lax.rsqrt(var + eps)
    y = xg.reshape(B, C, N)
    return y * gamma[None, :, None] + beta[None, :, None]


def _gn_nlc(x, gamma, beta):
    return jnp.transpose(_group_norm(jnp.transpose(x, (0, 2, 1)), gamma, beta), (0, 2, 1))


def _res_block(x, blk, up, lev):
    row, col, norm = lev
    x_un = _pool(x, up)
    h = jax.nn.relu(_gn_nlc(x_un, blk['gn1_g'], blk['gn1_b']))
    h = _cheb_conv(h, blk['c1'], row, col, norm)
    h = jax.nn.relu(_gn_nlc(h, blk['gn2_g'], blk['gn2_b']))
    h = _cheb_conv(h, blk['c2'], row, col, norm)
    h = jax.nn.relu(_gn_nlc(h, blk['gn3_g'], blk['gn3_b']))
    h = _cheb_conv(h, blk['c3'], row, col, norm)
    if 'c4' in blk:
        x_un = _cheb_conv(x_un, blk['c4'], row, col, norm)
    return h + x_un


def _identity_pallas(x):
    def body(x_ref, o_ref):
        o_ref[...] = x_ref[...]
    return pl.pallas_call(
        body, out_shape=jax.ShapeDtypeStruct(x.shape, x.dtype))(x)


def kernel(x, params, levels, ups):
    bs = x.shape[0]
    h = x @ params['fc1_w'] + params['fc1_b']
    h = jax.nn.leaky_relu(h, negative_slope=0.2)
    h = h.reshape(bs, _NUM_NODES[-1], -1)
    row, col, norm = levels[-1]
    h = _cheb_conv(h, params['conv1_w'], row, col, norm)
    for i in range(8):
        h = _res_block(h, params['blocks'][i], ups[-i - 1], levels[-i - 2])
    row, col, norm = levels[0]
    h = _cheb_conv(h, params['conv_out_w'], row, col, norm)
    out = h + params['out_bias']
    return _identity_pallas(out.reshape(16, 6890 * 3)).reshape(16, 6890, 3)


# speedup vs baseline: 3.4712x; 3.4712x over previous
import functools

import jax
import jax.numpy as jnp
from jax import lax
from jax.experimental import pallas as pl
from jax.experimental.pallas import tpu as pltpu
from jax.experimental.pallas import tpu_sc as plsc

_B = 16
_GROUPS = 32
_GN_EPS = 1e-5
_NNODES = [6890, 3445, 1723, 862, 431, 216, 108, 54, 27]


def _rup(x, m):
    return (x + m - 1) // m * m


def _fc1(x, w, b):
    m = w.shape[1]

    def body(x_ref, w_ref, b_ref, o_ref):
        h = jnp.dot(x_ref[...], w_ref[...], preferred_element_type=jnp.float32)
        h = h + b_ref[...]
        o_ref[...] = jnp.where(h >= 0, h, 0.2 * h)

    return pl.pallas_call(
        body,
        out_shape=jax.ShapeDtypeStruct((x.shape[0], m), jnp.float32),
    )(x, w, b.reshape(1, m))


def _batch_specs(shapes):
    specs = []
    for s in shapes:
        if len(s) == 3 and s[0] == _B:
            specs.append(pl.BlockSpec((1,) + s[1:], lambda i: (i, 0, 0)))
        elif len(s) == 3:
            specs.append(pl.BlockSpec((1,) + s[1:], lambda i: (0, 0, 0)))
        else:
            specs.append(pl.BlockSpec(s, lambda i, _n=len(s): (0,) * _n))
    return specs


def _tc_call(body, ins, out_c):
    np_ = ins[0].shape[1]
    out_shape = jax.ShapeDtypeStruct((_B, np_, out_c), jnp.float32)
    return pl.pallas_call(
        body,
        grid=(_B,),
        in_specs=_batch_specs([a.shape for a in ins]),
        out_specs=pl.BlockSpec((1, np_, out_c), lambda i: (i, 0, 0)),
        out_shape=out_shape,
    )(*ins)


def _gn(xs, n_valid, cin, np_, gamma, beta):
    cg = cin // _GROUPS
    rmask = lax.broadcasted_iota(jnp.int32, (np_, 1), 0) < n_valid
    xm = jnp.where(rmask, xs, 0.0)
    g1 = lax.broadcasted_iota(jnp.int32, (cin, _GROUPS), 0) // cg
    g2 = lax.broadcasted_iota(jnp.int32, (cin, _GROUPS), 1)
    G = (g1 == g2).astype(jnp.float32)
    cnt = float(n_valid * cg)
    s = jnp.sum(xm, axis=0, keepdims=True)
    mg = jnp.dot(s, G, preferred_element_type=jnp.float32,
                 precision=lax.Precision.HIGHEST) / cnt
    mean_c = jnp.dot(mg, G.T, preferred_element_type=jnp.float32,
                     precision=lax.Precision.HIGHEST)
    d = jnp.where(rmask, xs - mean_c, 0.0)
    ss = jnp.sum(d * d, axis=0, keepdims=True)
    vg = jnp.dot(ss, G, preferred_element_type=jnp.float32,
                 precision=lax.Precision.HIGHEST) / cnt
    sg = lax.rsqrt(vg + _GN_EPS)
    scale_c = jnp.dot(sg, G.T, preferred_element_type=jnp.float32,
                      precision=lax.Precision.HIGHEST)
    return d * scale_c * gamma + beta


def _gn_relu_mm_gn_relu(x, gamma1, beta1, w, gamma2, beta2, n_valid):
    _, np_, cin = x.shape
    cout = w.shape[1]

    def body(x_ref, g1_ref, b1_ref, w_ref, g2_ref, b2_ref, o_ref):
        y = _gn(x_ref[0], n_valid, cin, np_, g1_ref[...], b1_ref[...])
        y = jnp.maximum(y, 0.0)
        t = jnp.dot(y, w_ref[...], preferred_element_type=jnp.float32)
        h = _gn(t, n_valid, cout, np_, g2_ref[...], b2_ref[...])
        o_ref[...] = jnp.maximum(h, 0.0)[None]

    return _tc_call(
        body,
        [x, gamma1.reshape(1, cin), beta1.reshape(1, cin), w,
         gamma2.reshape(1, cout), beta2.reshape(1, cout)],
        cout,
    )


def _mm(x, w):
    _, np_, cin = x.shape
    cout = w.shape[1]

    def body(x_ref, w_ref, o_ref):
        o_ref[...] = jnp.dot(x_ref[0], w_ref[...],
                             preferred_element_type=jnp.float32)[None]

    return _tc_call(body, [x, w], cout)


def _cheb2_gn_relu_mm_add(h, tx1, w0, w1, gamma, beta, w3, skip, n_valid):
    _, np_, cin = h.shape
    cmid = w0.shape[1]
    cout = w3.shape[1]

    def body(h_ref, t_ref, w0_ref, w1_ref, g_ref, b_ref, w3_ref, s_ref, o_ref):
        t2 = (jnp.dot(h_ref[0], w0_ref[...], preferred_element_type=jnp.float32)
              + jnp.dot(t_ref[0], w1_ref[...], preferred_element_type=jnp.float32))
        y = _gn(t2, n_valid, cmid, np_, g_ref[...], b_ref[...])
        y = jnp.maximum(y, 0.0)
        o_ref[...] = (jnp.dot(y, w3_ref[...], preferred_element_type=jnp.float32)
                      + s_ref[0])[None]

    return _tc_call(
        body,
        [h, tx1, w0, w1, gamma.reshape(1, cmid), beta.reshape(1, cmid), w3, skip],
        cout,
    )


def _cheb2_bias(h, tx1, w0, w1, bias):
    _, np_, cin = h.shape
    cout = w0.shape[1]

    def body(h_ref, t_ref, w0_ref, w1_ref, b_ref, o_ref):
        o_ref[...] = (jnp.dot(h_ref[0], w0_ref[...], preferred_element_type=jnp.float32)
                      + jnp.dot(t_ref[0], w1_ref[...], preferred_element_type=jnp.float32)
                      + b_ref[0])[None]

    return _tc_call(body, [h, tx1, w0, w1, bias], cout)


_MESH = dict(core_axis_name="c", subcore_axis_name="s")


def _sc_unpool(x_flat, up_col, nin_pad, nout_pad, C):
    jpt = _rup(_rup(nout_pad, 32) // 32, 16)
    chunks = jpt // 16
    upc = jnp.pad(up_col.astype(jnp.int32), (0, 32 * jpt - up_col.shape[0]))

    @functools.partial(
        pl.kernel,
        mesh=plsc.VectorSubcoreMesh(**_MESH),
        compiler_params=pltpu.CompilerParams(
            needs_layout_passes=False, use_tc_tiling_on_sc=False),
        out_type=jax.ShapeDtypeStruct((_B * nout_pad, C), jnp.float32),
        scratch_types=[
            pltpu.VMEM((jpt,), jnp.int32),
            pltpu.VMEM((16, C), jnp.float32),
            pltpu.VMEM((16, C), jnp.float32),
            pltpu.VMEM((16, C), jnp.float32),
            pltpu.VMEM((16, C), jnp.float32),
            pltpu.SemaphoreType.DMA,
            pltpu.SemaphoreType.DMA,
            pltpu.SemaphoreType.DMA,
            pltpu.SemaphoreType.DMA,
        ],
    )
    def k(x_hbm, up_hbm, out_hbm, up_v, gb0, gb1, gb2, gb3,
          sem0, sem1, sem2, sem3):
        wid = lax.axis_index("s") * 2 + lax.axis_index("c")
        base_j = wid * jpt
        pltpu.sync_copy(up_hbm.at[pl.ds(base_j, jpt)], up_v)
        gbufs = (gb0, gb1, gb2, gb3)
        sems = (sem0, sem1, sem2, sem3)

        def chunk(g, carry):
            j0 = base_j + g * 16

            @pl.when(j0 < nout_pad)
            def _():
                up16 = up_v[pl.ds(g * 16, 16)]
                cps = [None, None, None, None]
                for b in range(3):
                    cps[b] = pltpu.async_copy(
                        x_hbm.at[up16 + b * nin_pad], gbufs[b], sems[b])
                for b in range(_B):
                    if b + 3 < _B:
                        cps[(b + 3) % 4] = pltpu.async_copy(
                            x_hbm.at[up16 + (b + 3) * nin_pad],
                            gbufs[(b + 3) % 4], sems[(b + 3) % 4])
                    cps[b % 4].wait()
                    pltpu.sync_copy(gbufs[b % 4],
                                    out_hbm.at[pl.ds(b * nout_pad + j0, 16)])
            return carry

        lax.fori_loop(0, chunks, chunk, 0)

    return k(x_flat, upc)


def _spec_mp(x_flat, row, col, norm, n_pad, C):
    x = x_flat.reshape(_B, n_pad, C)
    msg = x[:, col.astype(jnp.int32), :] * norm[None, :, None]
    return jnp.zeros_like(x).at[:, row.astype(jnp.int32), :].add(msg)


def _spec_unpool(x_flat, up_col, nin_pad, nout_pad, C):
    x = x_flat.reshape(_B, nin_pad, C)
    g = x[:, up_col.astype(jnp.int32), :]
    pad = nout_pad - g.shape[1]
    return jnp.pad(g, ((0, 0), (0, pad), (0, 0))).reshape(_B * nout_pad, C)


def _sc_unpool2(x_flat, up_col, nin_pad, nout_pad, C):
    return _spec_unpool(x_flat, up_col, nin_pad, nout_pad, C)


def _sc_mp(x_flat, row, col, norm, n_pad, C):
    return _sc_mp_real(x_flat, row, col, norm, n_pad, C)


def _sc_mp_real(x_flat, row, col, norm, n_pad, C):
    E = row.shape[0]
    Q = C // 16
    n_slabs = (16 * Q) // 32
    R = _B * n_pad
    nj = n_pad // 16

    W = 2048
    stage_src_pre = (2 * n_pad + 16) * 64 + 3 * 4 * W < 500_000
    e_pad = _rup(E, 256) if stage_src_pre else _rup(E, W)
    wins = []
    off = 0
    while off < e_pad:
        wins.append((off, min(W, e_pad - off)))
        off += W

    rowp = jnp.pad(row.astype(jnp.int32), (0, e_pad - E))
    colp = jnp.pad(col.astype(jnp.int32), (0, e_pad - E))
    normp = jnp.pad(norm.astype(jnp.float32), (0, e_pad - E))
    x_v = x_flat.reshape(R * Q, 16)

    stage_src = (2 * n_pad + 16) * 64 + 3 * 4 * W < 500_000

    scratch = [
        pltpu.VMEM((W,), jnp.int32),
        pltpu.VMEM((W,), jnp.int32),
        pltpu.VMEM((W,), jnp.float32),
        pltpu.VMEM((n_pad, 16), jnp.float32),
    ]
    if stage_src:
        scratch.append(pltpu.VMEM((n_pad, 16), jnp.float32))
    else:
        scratch.append(pltpu.VMEM((128, 16), jnp.float32))
        scratch.append(pltpu.VMEM((128, 16), jnp.float32))
        scratch.append(pltpu.VMEM((128,), jnp.int32))
        scratch.append(pltpu.VMEM((128,), jnp.int32))
    scratch += [pltpu.SemaphoreType.DMA, pltpu.SemaphoreType.DMA]

    @functools.partial(
        pl.kernel,
        mesh=plsc.VectorSubcoreMesh(**_MESH),
        compiler_params=pltpu.CompilerParams(
            needs_layout_passes=False, use_tc_tiling_on_sc=False),
        out_type=jax.ShapeDtypeStruct((R * Q, 16), jnp.float32),
        scratch_types=scratch,
    )
    def k(x_hbm, row_hbm, col_hbm, norm_hbm, out_hbm,
          erow, ecol, enorm, acc, *rest):
        if stage_src:
            src, sem0, sem1 = rest
        else:
            gb0, gb1, ix0, ix1, sem0, sem1 = rest
        wid = lax.axis_index("s") * 2 + lax.axis_index("c")
        i16 = lax.iota(jnp.int32, 16)
        cols = [jnp.full((16,), jj, jnp.int32) for jj in range(16)]
        zero16 = jnp.zeros((16,), jnp.float32)

        for slab in range(n_slabs):
            t = slab * 32 + wid
            b = t // Q
            q = t - b * Q
            xbase = b * n_pad * Q + q

            def zb(jg, carry):
                ridx = jg * 16 + i16
                for jj in range(16):
                    plsc.store_scatter(acc, [ridx, cols[jj]], zero16)
                return carry

            lax.fori_loop(0, nj, zb, 0)

            if stage_src:
                nfire = nj // 8
                rem = nj - nfire * 8

                def sb(g, carry):
                    cps = []
                    for u in range(8):
                        j0 = (g * 8 + u) * 16
                        idx = xbase + (j0 + i16) * Q
                        cps.append(pltpu.async_copy(
                            x_hbm.at[idx], src.at[pl.ds(j0, 16)], sem0))
                    for cp in cps:
                        cp.wait()
                    return carry

                lax.fori_loop(0, nfire, sb, 0)
                cps = []
                for u in range(rem):
                    j0 = (nfire * 8 + u) * 16
                    idx = xbase + (j0 + i16) * Q
                    cps.append(pltpu.async_copy(
                        x_hbm.at[idx], src.at[pl.ds(j0, 16)], sem0))
                for cp in cps:
                    cp.wait()

            if stage_src:
                for (woff, wlen) in wins:
                    pltpu.sync_copy(row_hbm.at[pl.ds(woff, wlen)],
                                    erow.at[pl.ds(0, wlen)])
                    pltpu.sync_copy(col_hbm.at[pl.ds(woff, wlen)],
                                    ecol.at[pl.ds(0, wlen)])
                    pltpu.sync_copy(norm_hbm.at[pl.ds(woff, wlen)],
                                    enorm.at[pl.ds(0, wlen)])

                    def cb(g, carry):
                        r16 = erow[pl.ds(g * 16, 16)]
                        c16 = ecol[pl.ds(g * 16, 16)]
                        n16 = enorm[pl.ds(g * 16, 16)]
                        for jj in range(16):
                            v = plsc.load_gather(src, [c16, cols[jj]])
                            plsc.addupdate_scatter(acc, [r16, cols[jj]],
                                                   v * n16)
                        return carry

                    lax.fori_loop(0, wlen // 16, cb, 0)
            else:
                gbufs = (gb0, gb1)
                ixbufs = (ix0, ix1)
                sems = (sem0, sem1)

                def win_body(w, carry):
                    woff = w * W
                    pltpu.sync_copy(row_hbm.at[pl.ds(woff, W)], erow)
                    pltpu.sync_copy(col_hbm.at[pl.ds(woff, W)], ecol)
                    pltpu.sync_copy(norm_hbm.at[pl.ds(woff, W)], enorm)

                    def cb(g2, carry2):
                        cps = [None, None]
                        for u in range(2):
                            base = (g2 * 2 + u) * 128
                            for k in range(8):
                                c16 = ecol[pl.ds(base + k * 16, 16)]
                                ixbufs[u][pl.ds(k * 16, 16)] = c16 * Q + xbase
                            cps[u] = pltpu.async_copy(
                                x_hbm.at[ixbufs[u]], gbufs[u], sems[u])
                        for u in range(2):
                            base = (g2 * 2 + u) * 128
                            cps[u].wait()
                            for k in range(8):
                                i16k = i16 + k * 16
                                r16 = erow[pl.ds(base + k * 16, 16)]
                                n16 = enorm[pl.ds(base + k * 16, 16)]
                                for jj in range(16):
                                    v = plsc.load_gather(gbufs[u],
                                                         [i16k, cols[jj]])
                                    plsc.addupdate_scatter(
                                        acc, [r16, cols[jj]], v * n16)
                        return carry2

                    lax.fori_loop(0, W // 256, cb, 0)
                    return carry

                lax.fori_loop(0, e_pad // W, win_body, 0)
            nfire = nj // 8
            rem = nj - nfire * 8

            def db(g, carry):
                cps = []
                for u in range(8):
                    j0 = (g * 8 + u) * 16
                    idx = xbase + (j0 + i16) * Q
                    cps.append(pltpu.async_copy(
                        acc.at[pl.ds(j0, 16)], out_hbm.at[idx], sem1))
                for cp in cps:
                    cp.wait()
                return carry

            lax.fori_loop(0, nfire, db, 0)
            cps = []
            for u in range(rem):
                j0 = (nfire * 8 + u) * 16
                idx = xbase + (j0 + i16) * Q
                cps.append(pltpu.async_copy(
                    acc.at[pl.ds(j0, 16)], out_hbm.at[idx], sem1))
            for cp in cps:
                cp.wait()

    out = k(x_v, rowp, colp, normp)
    return out.reshape(_B, n_pad, C)


def kernel(x, params, levels, ups):
    filters = [64, 64, 128, 128, 256, 256, 512, 512]
    res_dim = filters + [512]

    h = _fc1(x, params['fc1_w'], params['fc1_b'])
    h = h.reshape(_B, 27, 64)
    np_cur = _rup(27, 16)
    h = jnp.pad(h, ((0, 0), (0, np_cur - 27), (0, 0)))
    h = _mm(h, params['conv1_w'][0])

    for i in range(8):
        blk = params['blocks'][i]
        lev = levels[7 - i]
        up = ups[7 - i]
        n_out = _NNODES[7 - i]
        cin = res_dim[-i - 1]
        cout = res_dim[-i - 2]
        nin_pad = np_cur
        nout_pad = _rup(n_out, 16)

        x_un_f = _sc_unpool(h.reshape(_B * nin_pad, cin), up[1],
                            nin_pad, nout_pad, cin)
        x_un = x_un_f.reshape(_B, nout_pad, cin)

        hh = _gn_relu_mm_gn_relu(x_un, blk['gn1_g'], blk['gn1_b'],
                                 blk['c1'][0], blk['gn2_g'], blk['gn2_b'],
                                 n_out)
        tx1 = _sc_mp(hh.reshape(_B * nout_pad, cout // 2),
                     lev[0], lev[1], lev[2], nout_pad, cout // 2)
        skip = _mm(x_un, blk['c4'][0]) if 'c4' in blk else x_un
        h = _cheb2_gn_relu_mm_add(hh, tx1, blk['c2'][0], blk['c2'][1],
                                  blk['gn3_g'], blk['gn3_b'], blk['c3'][0],
                                  skip, n_out)
        np_cur = nout_pad

    lev = levels[0]
    tx1 = _sc_mp(h.reshape(_B * np_cur, 64), lev[0], lev[1], lev[2],
                 np_cur, 64)
    bias = jnp.pad(params['out_bias'], ((0, 0), (0, np_cur - 6890), (0, 0)))
    out = _cheb2_bias(h, tx1, params['conv_out_w'][0], params['conv_out_w'][1],
                      bias)
    return out[:, :6890, :]


# speedup vs baseline: 3.4726x; 1.0004x over previous
import functools

import jax
import jax.numpy as jnp
from jax import lax
from jax.experimental import pallas as pl
from jax.experimental.pallas import tpu as pltpu
from jax.experimental.pallas import tpu_sc as plsc

_B = 16
_GROUPS = 32
_GN_EPS = 1e-5
_NNODES = [6890, 3445, 1723, 862, 431, 216, 108, 54, 27]


def _rup(x, m):
    return (x + m - 1) // m * m


def _fc1(x, w, b):
    m = w.shape[1]

    def body(x_ref, w_ref, b_ref, o_ref):
        h = jnp.dot(x_ref[...], w_ref[...], preferred_element_type=jnp.float32)
        h = h + b_ref[...]
        o_ref[...] = jnp.where(h >= 0, h, 0.2 * h)

    return pl.pallas_call(
        body,
        out_shape=jax.ShapeDtypeStruct((x.shape[0], m), jnp.float32),
    )(x, w, b.reshape(1, m))


def _batch_specs(shapes):
    specs = []
    for s in shapes:
        if len(s) == 3 and s[0] == _B:
            specs.append(pl.BlockSpec((1,) + s[1:], lambda i: (i, 0, 0)))
        elif len(s) == 3:
            specs.append(pl.BlockSpec((1,) + s[1:], lambda i: (0, 0, 0)))
        else:
            specs.append(pl.BlockSpec(s, lambda i, _n=len(s): (0,) * _n))
    return specs


def _tc_call(body, ins, out_c):
    np_ = ins[0].shape[1]
    out_shape = jax.ShapeDtypeStruct((_B, np_, out_c), jnp.float32)
    return pl.pallas_call(
        body,
        grid=(_B,),
        in_specs=_batch_specs([a.shape for a in ins]),
        out_specs=pl.BlockSpec((1, np_, out_c), lambda i: (i, 0, 0)),
        out_shape=out_shape,
    )(*ins)


def _gn(xs, n_valid, cin, np_, gamma, beta):
    cg = cin // _GROUPS
    rmask = lax.broadcasted_iota(jnp.int32, (np_, 1), 0) < n_valid
    xm = jnp.where(rmask, xs, 0.0)
    g1 = lax.broadcasted_iota(jnp.int32, (cin, _GROUPS), 0) // cg
    g2 = lax.broadcasted_iota(jnp.int32, (cin, _GROUPS), 1)
    G = (g1 == g2).astype(jnp.float32)
    cnt = float(n_valid * cg)
    s = jnp.sum(xm, axis=0, keepdims=True)
    mg = jnp.dot(s, G, preferred_element_type=jnp.float32,
                 precision=lax.Precision.HIGHEST) / cnt
    mean_c = jnp.dot(mg, G.T, preferred_element_type=jnp.float32,
                     precision=lax.Precision.HIGHEST)
    d = jnp.where(rmask, xs - mean_c, 0.0)
    ss = jnp.sum(d * d, axis=0, keepdims=True)
    vg = jnp.dot(ss, G, preferred_element_type=jnp.float32,
                 precision=lax.Precision.HIGHEST) / cnt
    sg = lax.rsqrt(vg + _GN_EPS)
    scale_c = jnp.dot(sg, G.T, preferred_element_type=jnp.float32,
                      precision=lax.Precision.HIGHEST)
    return d * scale_c * gamma + beta


def _gn_relu_mm_gn_relu(x, gamma1, beta1, w, gamma2, beta2, n_valid):
    _, np_, cin = x.shape
    cout = w.shape[1]

    def body(x_ref, g1_ref, b1_ref, w_ref, g2_ref, b2_ref, o_ref):
        y = _gn(x_ref[0], n_valid, cin, np_, g1_ref[...], b1_ref[...])
        y = jnp.maximum(y, 0.0)
        t = jnp.dot(y, w_ref[...], preferred_element_type=jnp.float32)
        h = _gn(t, n_valid, cout, np_, g2_ref[...], b2_ref[...])
        o_ref[...] = jnp.maximum(h, 0.0)[None]

    return _tc_call(
        body,
        [x, gamma1.reshape(1, cin), beta1.reshape(1, cin), w,
         gamma2.reshape(1, cout), beta2.reshape(1, cout)],
        cout,
    )


def _mm(x, w):
    _, np_, cin = x.shape
    cout = w.shape[1]

    def body(x_ref, w_ref, o_ref):
        o_ref[...] = jnp.dot(x_ref[0], w_ref[...],
                             preferred_element_type=jnp.float32)[None]

    return _tc_call(body, [x, w], cout)


def _cheb2_gn_relu_mm_add(h, tx1, w0, w1, gamma, beta, w3, skip, n_valid):
    _, np_, cin = h.shape
    cmid = w0.shape[1]
    cout = w3.shape[1]

    def body(h_ref, t_ref, w0_ref, w1_ref, g_ref, b_ref, w3_ref, s_ref, o_ref):
        t2 = (jnp.dot(h_ref[0], w0_ref[...], preferred_element_type=jnp.float32)
              + jnp.dot(t_ref[0], w1_ref[...], preferred_element_type=jnp.float32))
        y = _gn(t2, n_valid, cmid, np_, g_ref[...], b_ref[...])
        y = jnp.maximum(y, 0.0)
        o_ref[...] = (jnp.dot(y, w3_ref[...], preferred_element_type=jnp.float32)
                      + s_ref[0])[None]

    return _tc_call(
        body,
        [h, tx1, w0, w1, gamma.reshape(1, cmid), beta.reshape(1, cmid), w3, skip],
        cout,
    )


def _cheb2_bias(h, tx1, w0, w1, bias):
    _, np_, cin = h.shape
    cout = w0.shape[1]

    def body(h_ref, t_ref, w0_ref, w1_ref, b_ref, o_ref):
        o_ref[...] = (jnp.dot(h_ref[0], w0_ref[...], preferred_element_type=jnp.float32)
                      + jnp.dot(t_ref[0], w1_ref[...], preferred_element_type=jnp.float32)
                      + b_ref[0])[None]

    return _tc_call(body, [h, tx1, w0, w1, bias], cout)


_MESH = dict(core_axis_name="c", subcore_axis_name="s")


def _sc_unpool(x_flat, up_col, nin_pad, nout_pad, C):
    jpt = _rup(_rup(nout_pad, 32) // 32, 16)
    chunks = jpt // 16
    upc = jnp.pad(up_col.astype(jnp.int32), (0, 32 * jpt - up_col.shape[0]))

    @functools.partial(
        pl.kernel,
        mesh=plsc.VectorSubcoreMesh(**_MESH),
        compiler_params=pltpu.CompilerParams(
            needs_layout_passes=False, use_tc_tiling_on_sc=False),
        out_type=jax.ShapeDtypeStruct((_B * nout_pad, C), jnp.float32),
        scratch_types=[
            pltpu.VMEM((jpt,), jnp.int32),
            pltpu.VMEM((16, C), jnp.float32),
            pltpu.VMEM((16, C), jnp.float32),
            pltpu.VMEM((16, C), jnp.float32),
            pltpu.VMEM((16, C), jnp.float32),
            pltpu.SemaphoreType.DMA,
            pltpu.SemaphoreType.DMA,
            pltpu.SemaphoreType.DMA,
            pltpu.SemaphoreType.DMA,
        ],
    )
    def k(x_hbm, up_hbm, out_hbm, up_v, gb0, gb1, gb2, gb3,
          sem0, sem1, sem2, sem3):
        wid = lax.axis_index("s") * 2 + lax.axis_index("c")
        base_j = wid * jpt
        pltpu.sync_copy(up_hbm.at[pl.ds(base_j, jpt)], up_v)
        gbufs = (gb0, gb1, gb2, gb3)
        sems = (sem0, sem1, sem2, sem3)

        def chunk(g, carry):
            j0 = base_j + g * 16

            @pl.when(j0 < nout_pad)
            def _():
                up16 = up_v[pl.ds(g * 16, 16)]
                cps = [None, None, None, None]
                for b in range(3):
                    cps[b] = pltpu.async_copy(
                        x_hbm.at[up16 + b * nin_pad], gbufs[b], sems[b])
                for b in range(_B):
                    if b + 3 < _B:
                        cps[(b + 3) % 4] = pltpu.async_copy(
                            x_hbm.at[up16 + (b + 3) * nin_pad],
                            gbufs[(b + 3) % 4], sems[(b + 3) % 4])
                    cps[b % 4].wait()
                    pltpu.sync_copy(gbufs[b % 4],
                                    out_hbm.at[pl.ds(b * nout_pad + j0, 16)])
            return carry

        lax.fori_loop(0, chunks, chunk, 0)

    return k(x_flat, upc)


def _sc_mp(x_flat, row, col, norm, n_pad, C):
    E = row.shape[0]
    Q = C // 16
    n_slabs = (16 * Q) // 32
    R = _B * n_pad
    nj = n_pad // 16

    W = 2048
    stage_src_pre = (2 * n_pad + 16) * 64 + 3 * 4 * W < 500_000
    e_pad = _rup(E, 256) if stage_src_pre else _rup(E, W)
    wins = []
    off = 0
    while off < e_pad:
        wins.append((off, min(W, e_pad - off)))
        off += W

    rowp = jnp.pad(row.astype(jnp.int32), (0, e_pad - E))
    colp = jnp.pad(col.astype(jnp.int32), (0, e_pad - E))
    normp = jnp.pad(norm.astype(jnp.float32), (0, e_pad - E))
    x_v = x_flat.reshape(R * Q, 16)

    stage_src = (2 * n_pad + 16) * 64 + 3 * 4 * W < 500_000

    scratch = [
        pltpu.VMEM((W,), jnp.int32),
        pltpu.VMEM((W,), jnp.int32),
        pltpu.VMEM((W,), jnp.float32),
        pltpu.VMEM((n_pad, 16), jnp.float32),
    ]
    if stage_src:
        scratch.append(pltpu.VMEM((n_pad, 16), jnp.float32))
    else:
        scratch.append(pltpu.VMEM((128, 16), jnp.float32))
        scratch.append(pltpu.VMEM((128, 16), jnp.float32))
        scratch.append(pltpu.VMEM((128,), jnp.int32))
        scratch.append(pltpu.VMEM((128,), jnp.int32))
    scratch += [pltpu.SemaphoreType.DMA, pltpu.SemaphoreType.DMA]

    @functools.partial(
        pl.kernel,
        mesh=plsc.VectorSubcoreMesh(**_MESH),
        compiler_params=pltpu.CompilerParams(
            needs_layout_passes=False, use_tc_tiling_on_sc=False),
        out_type=jax.ShapeDtypeStruct((R * Q, 16), jnp.float32),
        scratch_types=scratch,
    )
    def k(x_hbm, row_hbm, col_hbm, norm_hbm, out_hbm,
          erow, ecol, enorm, acc, *rest):
        if stage_src:
            src, sem0, sem1 = rest
        else:
            gb0, gb1, ix0, ix1, sem0, sem1 = rest
        wid = lax.axis_index("s") * 2 + lax.axis_index("c")
        i16 = lax.iota(jnp.int32, 16)
        cols = [jnp.full((16,), jj, jnp.int32) for jj in range(16)]
        zero16 = jnp.zeros((16,), jnp.float32)

        for slab in range(n_slabs):
            t = slab * 32 + wid
            b = t // Q
            q = t - b * Q
            xbase = b * n_pad * Q + q

            def zb(jg, carry):
                ridx = jg * 16 + i16
                for jj in range(16):
                    plsc.store_scatter(acc, [ridx, cols[jj]], zero16)
                return carry

            lax.fori_loop(0, nj, zb, 0)

            if stage_src:
                nfire = nj // 8
                rem = nj - nfire * 8

                def sb(g, carry):
                    cps = []
                    for u in range(8):
                        j0 = (g * 8 + u) * 16
                        idx = xbase + (j0 + i16) * Q
                        cps.append(pltpu.async_copy(
                            x_hbm.at[idx], src.at[pl.ds(j0, 16)], sem0))
                    for cp in cps:
                        cp.wait()
                    return carry

                lax.fori_loop(0, nfire, sb, 0)
                cps = []
                for u in range(rem):
                    j0 = (nfire * 8 + u) * 16
                    idx = xbase + (j0 + i16) * Q
                    cps.append(pltpu.async_copy(
                        x_hbm.at[idx], src.at[pl.ds(j0, 16)], sem0))
                for cp in cps:
                    cp.wait()

            if stage_src:
                for (woff, wlen) in wins:
                    pltpu.sync_copy(row_hbm.at[pl.ds(woff, wlen)],
                                    erow.at[pl.ds(0, wlen)])
                    pltpu.sync_copy(col_hbm.at[pl.ds(woff, wlen)],
                                    ecol.at[pl.ds(0, wlen)])
                    pltpu.sync_copy(norm_hbm.at[pl.ds(woff, wlen)],
                                    enorm.at[pl.ds(0, wlen)])

                    def cb(g, carry):
                        r16 = erow[pl.ds(g * 16, 16)]
                        c16 = ecol[pl.ds(g * 16, 16)]
                        n16 = enorm[pl.ds(g * 16, 16)]
                        for jj in range(16):
                            v = plsc.load_gather(src, [c16, cols[jj]])
                            plsc.addupdate_scatter(acc, [r16, cols[jj]],
                                                   v * n16)
                        return carry

                    lax.fori_loop(0, wlen // 16, cb, 0)
            else:
                gbufs = (gb0, gb1)
                ixbufs = (ix0, ix1)
                sems = (sem0, sem1)

                def win_body(w, carry):
                    woff = w * W
                    pltpu.sync_copy(row_hbm.at[pl.ds(woff, W)], erow)
                    pltpu.sync_copy(col_hbm.at[pl.ds(woff, W)], ecol)
                    pltpu.sync_copy(norm_hbm.at[pl.ds(woff, W)], enorm)

                    def cb(g2, carry2):
                        cps = [None, None]
                        for u in range(2):
                            base = (g2 * 2 + u) * 128
                            for k in range(8):
                                c16 = ecol[pl.ds(base + k * 16, 16)]
                                ixbufs[u][pl.ds(k * 16, 16)] = c16 * Q + xbase
                            cps[u] = pltpu.async_copy(
                                x_hbm.at[ixbufs[u]], gbufs[u], sems[u])
                        for u in range(2):
                            base = (g2 * 2 + u) * 128
                            cps[u].wait()
                            for k in range(8):
                                i16k = i16 + k * 16
                                r16 = erow[pl.ds(base + k * 16, 16)]
                                n16 = enorm[pl.ds(base + k * 16, 16)]
                                for jj in range(16):
                                    v = plsc.load_gather(gbufs[u],
                                                         [i16k, cols[jj]])
                                    plsc.addupdate_scatter(
                                        acc, [r16, cols[jj]], v * n16)
                        return carry2

                    lax.fori_loop(0, W // 256, cb, 0)
                    return carry

                lax.fori_loop(0, e_pad // W, win_body, 0)
            nfire = nj // 8
            rem = nj - nfire * 8

            def db(g, carry):
                cps = []
                for u in range(8):
                    j0 = (g * 8 + u) * 16
                    idx = xbase + (j0 + i16) * Q
                    cps.append(pltpu.async_copy(
                        acc.at[pl.ds(j0, 16)], out_hbm.at[idx], sem1))
                for cp in cps:
                    cp.wait()
                return carry

            lax.fori_loop(0, nfire, db, 0)
            cps = []
            for u in range(rem):
                j0 = (nfire * 8 + u) * 16
                idx = xbase + (j0 + i16) * Q
                cps.append(pltpu.async_copy(
                    acc.at[pl.ds(j0, 16)], out_hbm.at[idx], sem1))
            for cp in cps:
                cp.wait()

    out = k(x_v, rowp, colp, normp)
    return out.reshape(_B, n_pad, C)


def kernel(x, params, levels, ups):
    filters = [64, 64, 128, 128, 256, 256, 512, 512]
    res_dim = filters + [512]

    h = _fc1(x, params['fc1_w'], params['fc1_b'])
    h = h.reshape(_B, 27, 64)
    np_cur = _rup(27, 16)
    h = jnp.pad(h, ((0, 0), (0, np_cur - 27), (0, 0)))
    h = _mm(h, params['conv1_w'][0])

    for i in range(8):
        blk = params['blocks'][i]
        lev = levels[7 - i]
        up = ups[7 - i]
        n_out = _NNODES[7 - i]
        cin = res_dim[-i - 1]
        cout = res_dim[-i - 2]
        nin_pad = np_cur
        nout_pad = _rup(n_out, 16)

        x_un_f = _sc_unpool(h.reshape(_B * nin_pad, cin), up[1],
                            nin_pad, nout_pad, cin)
        x_un = x_un_f.reshape(_B, nout_pad, cin)

        hh = _gn_relu_mm_gn_relu(x_un, blk['gn1_g'], blk['gn1_b'],
                                 blk['c1'][0], blk['gn2_g'], blk['gn2_b'],
                                 n_out)
        tx1 = _sc_mp(hh.reshape(_B * nout_pad, cout // 2),
                     lev[0], lev[1], lev[2], nout_pad, cout // 2)
        skip = _mm(x_un, blk['c4'][0]) if 'c4' in blk else x_un
        h = _cheb2_gn_relu_mm_add(hh, tx1, blk['c2'][0], blk['c2'][1],
                                  blk['gn3_g'], blk['gn3_b'], blk['c3'][0],
                                  skip, n_out)
        np_cur = nout_pad

    lev = levels[0]
    tx1 = _sc_mp(h.reshape(_B * np_cur, 64), lev[0], lev[1], lev[2],
                 np_cur, 64)
    bias = jnp.pad(params['out_bias'], ((0, 0), (0, np_cur - 6890), (0, 0)))
    out = _cheb2_bias(h, tx1, params['conv_out_w'][0], params['conv_out_w'][1],
                      bias)
    return out[:, :6890, :]
